# baseline (speedup 1.0000x reference)
import functools

import jax
import jax.numpy as jnp
from jax import lax
from jax.experimental import pallas as pl
from jax.experimental.pallas import tpu as pltpu
from jax.experimental.pallas import tpu_sc as plsc

_P = 8
_DP = 96
_AD = 8
_ADP = 16
_H = 64
_LD = _P * _DP
_BLK = 512
_NW = 32
_CH = 128


def _routing_metadata(pol, n_pad, n_blk):
    n = pol.shape[0]
    oh = (pol[:, None] == jnp.arange(_P, dtype=jnp.int32)[None, :]).astype(jnp.int32)
    rank = jnp.take_along_axis(jnp.cumsum(oh, axis=0) - oh, pol[:, None], axis=1)[:, 0]
    counts = jnp.sum(oh, axis=0)
    padded = ((counts + _BLK - 1) // _BLK) * _BLK
    seg_end = jnp.cumsum(padded)
    seg_start = seg_end - padded
    slot = seg_start[pol] + rank
    gidx = jnp.zeros((n_pad,), jnp.int32).at[slot].set(
        jnp.arange(n, dtype=jnp.int32))
    bpol = jnp.searchsorted(
        seg_end, jnp.arange(n_blk, dtype=jnp.int32) * _BLK, side="right")
    bpol = jnp.minimum(bpol, _P - 1).astype(jnp.int32)
    return slot, gidx, bpol


def _pack_weights(W1, b1, W2, b2):
    Wz = W1[:, :, :_DP, :]
    W1z = jnp.zeros((_P, 4, 2 * _DP, 2 * _H), W1.dtype)
    W1z = W1z.at[:, :, :_DP, :_H].set(Wz[:, 0::2])
    W1z = W1z.at[:, :, _DP:, _H:].set(Wz[:, 1::2])
    Wa = jnp.transpose(W1[:, :, _DP:, :], (0, 2, 1, 3)).reshape(_P, _AD, _P * _H)
    W1a = jnp.zeros((_P, _ADP, _P * _H), W1.dtype).at[:, :_AD, :].set(Wa)
    b1f = b1.reshape(_P, 1, _P * _H)
    W2p = jnp.zeros((_P, 4, 2 * _H, 2 * _DP), W2.dtype)
    W2p = W2p.at[:, :, :_H, :_DP].set(W2[:, 0::2])
    W2p = W2p.at[:, :, _H:, _DP:].set(W2[:, 1::2])
    b2f = b2.reshape(_P, 1, _P * _DP)
    return W1z, W1a, b1f, W2p, b2f


def _sc_mesh():
    return plsc.VectorSubcoreMesh(core_axis_name="c", subcore_axis_name="s")


def _gather_sorted(latents, actions_p, gidx, n_pad):
    rows_per_w = n_pad // _NW
    n_ch = rows_per_w // _CH

    @functools.partial(
        pl.kernel,
        out_type=(
            jax.ShapeDtypeStruct((n_pad, _LD), jnp.float32),
            jax.ShapeDtypeStruct((n_pad, _ADP), jnp.float32),
        ),
        mesh=_sc_mesh(),
        scratch_types=[
            pltpu.VMEM((_CH,), jnp.int32),
            pltpu.VMEM((_CH, _LD), jnp.float32),
            pltpu.VMEM((_CH, _ADP), jnp.float32),
            pltpu.SemaphoreType.DMA,
            pltpu.SemaphoreType.DMA,
        ],
        compiler_params=pltpu.CompilerParams(use_tc_tiling_on_sc=False),
    )
    def gather_k(lat_hbm, act_hbm, gidx_hbm, xg_hbm, ag_hbm,
                 idx_v, xrows_v, arows_v, sem_x, sem_a):
        wid = lax.axis_index("s") * 2 + lax.axis_index("c")
        for c in range(n_ch):
            base = wid * rows_per_w + c * _CH
            pltpu.sync_copy(gidx_hbm.at[pl.ds(base, _CH)], idx_v)
            cp_x = pltpu.async_copy(lat_hbm.at[idx_v], xrows_v, sem_x)
            cp_a = pltpu.async_copy(act_hbm.at[idx_v], arows_v, sem_a)
            cp_x.wait()
            cp_a.wait()
            pltpu.sync_copy(xrows_v, xg_hbm.at[pl.ds(base, _CH)])
            pltpu.sync_copy(arows_v, ag_hbm.at[pl.ds(base, _CH)])

    return gather_k(latents, actions_p, gidx)


def _gather_back(pad_out, slot, n):
    rows_per_w = n // _NW
    n_ch = rows_per_w // _CH

    @functools.partial(
        pl.kernel,
        out_type=jax.ShapeDtypeStruct((n, _LD), jnp.float32),
        mesh=_sc_mesh(),
        scratch_types=[
            pltpu.VMEM((_CH,), jnp.int32),
            pltpu.VMEM((_CH, _LD), jnp.float32),
            pltpu.SemaphoreType.DMA,
        ],
    )
    def back_k(pad_hbm, slot_hbm, out_hbm, idx_v, rows_v, sem):
        wid = lax.axis_index("s") * 2 + lax.axis_index("c")
        for c in range(n_ch):
            base = wid * rows_per_w + c * _CH
            pltpu.sync_copy(slot_hbm.at[pl.ds(base, _CH)], idx_v)
            pltpu.async_copy(pad_hbm.at[idx_v], rows_v, sem).wait()
            pltpu.sync_copy(rows_v, out_hbm.at[pl.ds(base, _CH)])

    return back_k(pad_out, slot)


def _mlp_body(bp_ref, x_ref, a_ref, w1z_ref, w1a_ref, b1_ref, w2_ref, b2_ref,
              o_ref):
    x = x_ref[...]
    a = a_ref[...]
    aterm = jnp.dot(a, w1a_ref[0], preferred_element_type=jnp.float32)
    for q in range(4):
        z = x[:, q * 192:(q + 1) * 192]
        h = jnp.dot(z, w1z_ref[0, q], preferred_element_type=jnp.float32)
        h = h + aterm[:, q * 128:(q + 1) * 128] + b1_ref[0, 0, q * 128:(q + 1) * 128]
        h = jnp.maximum(h, 0.0)
        y = jnp.dot(h, w2_ref[0, q], preferred_element_type=jnp.float32)
        o_ref[:, q * 192:(q + 1) * 192] = y + b2_ref[0, 0, q * 192:(q + 1) * 192]


def _mlp_blocks(xg, ag, bpol, W1z, W1a, b1f, W2p, b2f, n_pad, n_blk):
    grid_spec = pltpu.PrefetchScalarGridSpec(
        num_scalar_prefetch=1,
        grid=(n_blk,),
        in_specs=[
            pl.BlockSpec((_BLK, _LD), lambda k, bp: (k, 0)),
            pl.BlockSpec((_BLK, _ADP), lambda k, bp: (k, 0)),
            pl.BlockSpec((1, 4, 192, 128), lambda k, bp: (bp[k], 0, 0, 0)),
            pl.BlockSpec((1, _ADP, 512), lambda k, bp: (bp[k], 0, 0)),
            pl.BlockSpec((1, 1, 512), lambda k, bp: (bp[k], 0, 0)),
            pl.BlockSpec((1, 4, 128, 192), lambda k, bp: (bp[k], 0, 0, 0)),
            pl.BlockSpec((1, 1, 768), lambda k, bp: (bp[k], 0, 0)),
        ],
        out_specs=pl.BlockSpec((_BLK, _LD), lambda k, bp: (k, 0)),
    )
    return pl.pallas_call(
        _mlp_body,
        grid_spec=grid_spec,
        out_shape=jax.ShapeDtypeStruct((n_pad, _LD), jnp.float32),
    )(bpol, xg, ag, W1z, W1a, b1f, W2p, b2f)


def kernel(latents, policy_indices, actions, W1, b1, W2, b2):
    n = latents.shape[0]
    n_blk = n // _BLK + _P
    n_pad = n_blk * _BLK

    pol = policy_indices.astype(jnp.int32)
    slot, gidx, bpol = _routing_metadata(pol, n_pad, n_blk)
    W1z, W1a, b1f, W2p, b2f = _pack_weights(W1, b1, W2, b2)
    actions_p = jnp.zeros((n, _ADP), actions.dtype).at[:, :_AD].set(actions)

    xg, ag = _gather_sorted(latents, actions_p, gidx, n_pad)
    pad_out = _mlp_blocks(xg, ag, bpol, W1z, W1a, b1f, W2p, b2f, n_pad, n_blk)
    return _gather_back(pad_out, slot, n)
